# baseline (speedup 1.0000x reference)
import functools

import jax
import jax.numpy as jnp
from jax import lax
from jax.experimental import pallas as pl
from jax.experimental.pallas import tpu as pltpu
from jax.experimental.pallas import tpu_sc as plsc

_N_FRAMES = 8


def kernel(x, number):
    del number
    B, F, D = x.shape
    n = _N_FRAMES
    step = F // (n + 1)

    info = plsc.get_sparse_core_info()
    NC = info.num_cores
    rows_total = B * n
    rpw = 16
    nw = rows_total // rpw

    x2d = x.reshape(B * F, D)

    mesh = plsc.VectorSubcoreMesh(core_axis_name="c", subcore_axis_name="s")

    @functools.partial(
        pl.kernel,
        mesh=mesh,
        out_type=jax.ShapeDtypeStruct((rows_total, D), x.dtype),
        scratch_types=[
            pltpu.VMEM((rpw, D), x.dtype),
            pltpu.SemaphoreType.DMA,
        ],
    )
    def gather_rows(table_hbm, out_hbm, rows_v, sem):
        c = lax.axis_index("c")
        s = lax.axis_index("s")
        w = c * (nw // NC) + s

        @pl.when(s < nw // NC)
        def _():
            base = w * rpw
            r = base + lax.iota(jnp.int32, 16)
            b = r >> 3
            j = (r & 7) + 1
            idx = b * F + j * step
            pltpu.async_copy(table_hbm.at[idx], rows_v, sem).wait()
            pltpu.sync_copy(rows_v, out_hbm.at[pl.ds(base, rpw)])

    return gather_rows(x2d).reshape(B, n, D)


# speedup vs baseline: 1.0791x; 1.0791x over previous
import functools

import jax
import jax.numpy as jnp
from jax import lax
from jax.experimental import pallas as pl
from jax.experimental.pallas import tpu as pltpu
from jax.experimental.pallas import tpu_sc as plsc

_N_FRAMES = 8


def kernel(x, number):
    del number
    B, F, D = x.shape
    n = _N_FRAMES
    step = F // (n + 1)

    info = plsc.get_sparse_core_info()
    NS = info.num_subcores
    rows_total = B * n
    rpw = rows_total // NS
    half = rpw // 2

    x2d = x.reshape(B * F, D)

    mesh = plsc.VectorSubcoreMesh(
        core_axis_name="c", subcore_axis_name="s", num_cores=1
    )

    @functools.partial(
        pl.kernel,
        mesh=mesh,
        out_type=jax.ShapeDtypeStruct((rows_total, D), x.dtype),
        scratch_types=[
            pltpu.VMEM((rpw,), jnp.int32),
            pltpu.VMEM((rpw, D), x.dtype),
            pltpu.SemaphoreType.DMA,
            pltpu.SemaphoreType.DMA,
            pltpu.SemaphoreType.DMA,
        ],
    )
    def gather_rows(table_hbm, out_hbm, idx_v, rows_v, g_sem, h_sem, w_sem):
        w = lax.axis_index("s")
        base = w * rpw
        r = base + lax.iota(jnp.int32, 16)
        b = r >> 3
        j = (r & 7) + 1
        idx_v[...] = b * F + j * step
        g0 = pltpu.make_async_copy(
            table_hbm.at[idx_v.at[pl.ds(0, half)]], rows_v.at[pl.ds(0, half)],
            g_sem)
        g1 = pltpu.make_async_copy(
            table_hbm.at[idx_v.at[pl.ds(half, half)]],
            rows_v.at[pl.ds(half, half)], h_sem)
        g0.start()
        g1.start()
        g0.wait()
        wb0 = pltpu.make_async_copy(
            rows_v.at[pl.ds(0, half)], out_hbm.at[pl.ds(base, half)], w_sem)
        wb0.start()
        g1.wait()
        wb1 = pltpu.make_async_copy(
            rows_v.at[pl.ds(half, half)],
            out_hbm.at[pl.ds(base + half, half)], w_sem)
        wb1.start()
        wb0.wait()
        wb1.wait()

    return gather_rows(x2d).reshape(B, n, D)
